# Initial kernel scaffold; baseline (speedup 1.0000x reference)
import jax
import jax.numpy as jnp
from jax.experimental import pallas as pl


def kernel(obs, reward):
    raise NotImplementedError("write your pallas kernel here")


# speedup vs baseline: 2.2378x; 2.2378x over previous
import functools

import numpy as np
import jax
import jax.numpy as jnp
from jax import lax
from jax.experimental import pallas as pl
from jax.experimental.pallas import tpu as pltpu


def _round_up(x, m):
    return ((x + m - 1) // m) * m


def _ret_kernel(disc_ref, rew_ref, out_ref):
    out_ref[...] = jnp.dot(disc_ref[...], rew_ref[...],
                           preferred_element_type=jnp.float32,
                           precision=lax.Precision.HIGHEST)


def _gram_kernel(obs_ref, y_ref, out_ref, acc_ref, x_ref, *,
                 d, bsz, tile_r, k2, cmat, mrows):
    p = pl.program_id(0)
    k = pl.program_id(1)

    @pl.when(k == 0)
    def _init():
        acc_ref[...] = jnp.zeros_like(acc_ref)

    obs = obs_ref[...]
    x_ref[:, 0:d] = obs
    x_ref[:, d:2 * d] = obs * obs

    row0 = (p * k2 + k) * tile_r
    rowi = row0 + lax.broadcasted_iota(jnp.int32, (tile_r, 1), 0)
    t = (rowi // bsz).astype(jnp.float32) * 0.01
    t2 = t * t
    t3 = t2 * t
    lane = lax.broadcasted_iota(jnp.int32, (tile_r, 128), 1)
    blk = jnp.zeros((tile_r, 128), jnp.float32)
    for j in range(4):
        pj = cmat[0][j] + cmat[1][j] * t + cmat[2][j] * t2 + cmat[3][j] * t3
        blk = jnp.where(lane == j, pj, blk)
    blk = jnp.where(lane == 4, y_ref[...], blk)
    x_ref[:, 2 * d:2 * d + 128] = blk

    x = x_ref[...]
    acc_ref[...] += lax.dot_general(
        x_ref[:, 0:mrows], x, (((0,), (0,)), ((), ())),
        preferred_element_type=jnp.float32,
        precision=lax.Precision.HIGHEST)

    @pl.when(k == k2 - 1)
    def _done():
        out_ref[...] = acc_ref[...]


def _value_kernel(obs_ref, w_ref, out_ref, *, bsz, tile_r):
    i = pl.program_id(0)
    obs = obs_ref[...]
    wa = w_ref[0:1, :]
    wb = w_ref[1:2, :]
    s = obs * (wa + obs * wb)
    v = jnp.sum(s, axis=1, keepdims=True)
    rowi = i * tile_r + lax.broadcasted_iota(jnp.int32, (tile_r, 1), 0)
    t = (rowi // bsz).astype(jnp.float32) * 0.01
    q0 = w_ref[2:3, 0:1]
    q1 = w_ref[2:3, 1:2]
    q2 = w_ref[2:3, 2:3]
    q3 = w_ref[2:3, 3:4]
    out_ref[...] = v + (q0 + t * (q1 + t * (q2 + t * q3)))


def _ridge_solve(xtx, xty, reg0):
    n = xtx.shape[0]
    eye = jnp.eye(n, dtype=jnp.float32)

    def cond_fn(st):
        i, _, done = st
        return jnp.logical_and(jnp.logical_not(done), i < 5)

    def body_fn(st):
        i, w, _ = st
        reg = reg0 * lax.pow(10.0, i.astype(jnp.float32))
        with jax.default_matmul_precision("highest"):
            c = jax.scipy.linalg.cho_factor(xtx + reg * eye)
            wi = jax.scipy.linalg.cho_solve(c, xty)
        ok = jnp.all(jnp.isfinite(wi))
        return i + jnp.int32(1), jnp.where(ok, wi, w), ok

    _, w, _ = lax.while_loop(
        cond_fn, body_fn,
        (jnp.int32(0), jnp.zeros((n,), jnp.float32), jnp.bool_(False)))
    return w


def kernel(obs, reward):
    S, B, D = obs.shape
    N = S * B
    F = 2 * D + 4
    XP = _round_up(2 * D + 5, 128)
    MR = _round_up(2 * D + 5, 8)
    obs2d = obs.reshape(N, D)

    cmat = np.array([[0.0, 0.0, 0.0, 1.0],
                     [1.0, 0.0, 0.0, 0.0],
                     [0.0, 1.0, 0.0, 0.0],
                     [0.0, 0.0, 1.0, 0.0]])
    cmat_py = [[float(cmat[i, j]) for j in range(4)] for i in range(4)]
    ii = np.arange(S)
    disc_u = np.where(ii[:, None] <= ii[None, :],
                      np.power(0.99, ii[None, :] - ii[:, None]), 0.0).astype(np.float32)

    y2d = pl.pallas_call(
        _ret_kernel,
        out_shape=jax.ShapeDtypeStruct((S, B), jnp.float32),
    )(jnp.asarray(disc_u), reward)
    y_col = y2d.T.reshape(N, 1)

    tile_r = 8
    for cand in (4096, 2048, 1024, 512, 256, 128, 64, 32, 16, 8):
        if N % cand == 0:
            tile_r = cand
            break
    nt = N // tile_r
    p_split = 2 if nt % 2 == 0 else 1
    k2 = nt // p_split

    gram_parts = pl.pallas_call(
        functools.partial(_gram_kernel, d=D, bsz=B, tile_r=tile_r, k2=k2,
                          cmat=cmat_py, mrows=MR),
        out_shape=jax.ShapeDtypeStruct((p_split, MR, XP), jnp.float32),
        grid_spec=pltpu.PrefetchScalarGridSpec(
            num_scalar_prefetch=0,
            grid=(p_split, k2),
            in_specs=[
                pl.BlockSpec((tile_r, D), lambda p, k: (p * k2 + k, 0)),
                pl.BlockSpec((tile_r, 1), lambda p, k: (p * k2 + k, 0)),
            ],
            out_specs=pl.BlockSpec((None, MR, XP), lambda p, k: (p, 0, 0)),
            scratch_shapes=[pltpu.VMEM((MR, XP), jnp.float32),
                            pltpu.VMEM((tile_r, XP), jnp.float32)]),
        compiler_params=pltpu.CompilerParams(
            dimension_semantics=("parallel", "arbitrary")),
    )(obs2d, y_col)

    gram = jnp.sum(gram_parts, axis=0) if p_split > 1 else gram_parts[0]
    xtx = gram[:F, :F]
    xty = gram[:F, F]
    w = _ridge_solve(xtx, xty, 1e-5)

    qc = jnp.asarray(cmat, jnp.float32) @ w[2 * D:2 * D + 4]
    wmat = jnp.zeros((8, 128), jnp.float32)
    wmat = wmat.at[0, :D].set(w[:D]).at[1, :D].set(w[D:2 * D])
    wmat = wmat.at[2, :4].set(qc)

    val = pl.pallas_call(
        functools.partial(_value_kernel, bsz=B, tile_r=tile_r),
        out_shape=jax.ShapeDtypeStruct((N, 1), jnp.float32),
        grid_spec=pltpu.PrefetchScalarGridSpec(
            num_scalar_prefetch=0,
            grid=(nt,),
            in_specs=[pl.BlockSpec((tile_r, D), lambda i: (i, 0)),
                      pl.BlockSpec((8, 128), lambda i: (0, 0))],
            out_specs=pl.BlockSpec((tile_r, 1), lambda i: (i, 0))),
        compiler_params=pltpu.CompilerParams(
            dimension_semantics=("parallel",)),
    )(obs2d, wmat)

    return val.reshape(S, B).T


# speedup vs baseline: 2.7393x; 1.2241x over previous
import functools

import numpy as np
import jax
import jax.numpy as jnp
from jax import lax
from jax.experimental import pallas as pl
from jax.experimental.pallas import tpu as pltpu


def _round_up(x, m):
    return ((x + m - 1) // m) * m


def _returns_kernel(rew_ref, disc_ref, ret_ref, acc_ref):
    j = pl.program_id(1)
    k = pl.program_id(2)

    @pl.when(k == 0)
    def _init():
        acc_ref[...] = jnp.zeros_like(acc_ref)

    @pl.when(k >= j)
    def _acc():
        acc_ref[...] += jnp.dot(rew_ref[...], disc_ref[...],
                                preferred_element_type=jnp.float32,
                                precision=lax.Precision.HIGHEST)

    @pl.when(k == pl.num_programs(2) - 1)
    def _done():
        ret_ref[...] = acc_ref[...]


def _gram_kernel(obs_ref, y_ref, gram_ref, acc_ref, feat_ref, *,
                 d, n_rows, batch_size, tile_n):
    p = pl.program_id(0)
    k = pl.program_id(1)

    @pl.when(k == 0)
    def _init():
        acc_ref[...] = jnp.zeros_like(acc_ref)

    col_start = (p * pl.num_programs(1) + k) * tile_n
    lane = lax.broadcasted_iota(jnp.int32, (1, tile_n), 1)
    col = col_start + lane
    vmask = col < n_rows
    obs = jnp.where(vmask, obs_ref[...].T, 0.0)
    t = jnp.where(vmask, (col // batch_size).astype(jnp.float32) * 0.01, 0.0)
    feat_ref[0:d, :] = obs
    feat_ref[d:2 * d, :] = obs * obs
    feat_ref[2 * d:2 * d + 1, :] = t
    feat_ref[2 * d + 1:2 * d + 2, :] = t * t
    feat_ref[2 * d + 2:2 * d + 3, :] = t * t * t
    feat_ref[2 * d + 3:2 * d + 4, :] = vmask.astype(jnp.float32)
    feat_ref[2 * d + 4:2 * d + 5, :] = jnp.where(vmask, y_ref[...], 0.0)

    feat = feat_ref[...]
    acc_ref[...] += lax.dot_general(
        feat, feat, (((1,), (1,)), ((), ())),
        preferred_element_type=jnp.float32, precision=lax.Precision.HIGHEST)

    @pl.when(k == pl.num_programs(1) - 1)
    def _done():
        gram_ref[...] = acc_ref[...]


def _value_kernel(obs_ref, w_ref, out_ref, *, bsz, tile_r):
    i = pl.program_id(0)
    obs = obs_ref[...]
    wa = w_ref[0:1, :]
    wb = w_ref[1:2, :]
    s = obs * (wa + obs * wb)
    v = jnp.sum(s, axis=1, keepdims=True)
    rowi = i * tile_r + lax.broadcasted_iota(jnp.int32, (tile_r, 1), 0)
    t = (rowi // bsz).astype(jnp.float32) * 0.01
    q0 = w_ref[2:3, 0:1]
    q1 = w_ref[2:3, 1:2]
    q2 = w_ref[2:3, 2:3]
    q3 = w_ref[2:3, 3:4]
    out_ref[...] = v + (q0 + t * (q1 + t * (q2 + t * q3)))


def _ridge_solve(xtx, xty):
    n = xtx.shape[0]
    eye = jnp.eye(n, dtype=jnp.float32)
    regs = jnp.asarray([1e-5, 1e-4, 1e-3, 1e-2, 1e-1], jnp.float32)

    def cond_fn(st):
        i, _, done = st
        return jnp.logical_and(jnp.logical_not(done), i < 5)

    def body_fn(st):
        i, w, _ = st
        c = jax.scipy.linalg.cho_factor(xtx + regs[i] * eye)
        wi = jax.scipy.linalg.cho_solve(c, xty)
        ok = jnp.all(jnp.isfinite(wi))
        return i + jnp.int32(1), jnp.where(ok, wi, w), ok

    _, w, _ = lax.while_loop(
        cond_fn, body_fn,
        (jnp.int32(0), jnp.zeros((n,), jnp.float32), jnp.bool_(False)))
    return w


def kernel(obs, reward):
    S, B, D = obs.shape
    N = S * B
    F = 2 * D + 4
    Fa = F + 1
    obs2d = obs.reshape(N, D)

    rew = reward.T
    tile_b = _round_up(min(256, _round_up(B, 8)), 8)
    tile_s = _round_up(min(512, _round_up(S, 128)), 128)
    Bp, Sp = _round_up(B, tile_b), _round_up(S, tile_s)
    rew_p = jnp.zeros((Bp, Sp), jnp.float32).at[:B, :S].set(rew)
    idx = np.arange(Sp)
    dk = np.maximum(idx[:, None] - idx[None, :], 0)
    disc = np.where(idx[:, None] >= idx[None, :],
                    np.power(0.99, dk), 0.0).astype(np.float32)

    ret = pl.pallas_call(
        _returns_kernel,
        out_shape=jax.ShapeDtypeStruct((Bp, Sp), jnp.float32),
        grid_spec=pltpu.PrefetchScalarGridSpec(
            num_scalar_prefetch=0,
            grid=(Bp // tile_b, Sp // tile_s, Sp // tile_s),
            in_specs=[
                pl.BlockSpec((tile_b, tile_s),
                             lambda b, j, k: (b, jnp.maximum(k, j))),
                pl.BlockSpec((tile_s, tile_s),
                             lambda b, j, k: (jnp.maximum(k, j), j)),
            ],
            out_specs=pl.BlockSpec((tile_b, tile_s), lambda b, j, k: (b, j)),
            scratch_shapes=[pltpu.VMEM((tile_b, tile_s), jnp.float32)]),
        compiler_params=pltpu.CompilerParams(
            dimension_semantics=("parallel", "parallel", "arbitrary")),
    )(rew_p, jnp.asarray(disc))[:B, :S]
    y_row = ret.reshape(1, N)

    tile_n = _round_up(min(8192, _round_up(N, 128)), 128)
    per_128 = (2 * _round_up(D, 8) + 2 * 8 + _round_up(Fa, 8)) * 128 * 4
    max_tile = max(128, (10 * 1024 * 1024 // per_128) * 128)
    tile_n = min(tile_n, max_tile)
    nt = pl.cdiv(N, tile_n)
    P = max(1, min(2, nt))
    tpp = pl.cdiv(nt, P)

    gram_parts = pl.pallas_call(
        functools.partial(_gram_kernel, d=D, n_rows=N, batch_size=B,
                          tile_n=tile_n),
        out_shape=jax.ShapeDtypeStruct((P, Fa, Fa), jnp.float32),
        grid_spec=pltpu.PrefetchScalarGridSpec(
            num_scalar_prefetch=0,
            grid=(P, tpp),
            in_specs=[
                pl.BlockSpec((tile_n, D),
                             lambda p, k: (jnp.minimum(p * tpp + k, nt - 1), 0)),
                pl.BlockSpec((1, tile_n),
                             lambda p, k: (0, jnp.minimum(p * tpp + k, nt - 1))),
            ],
            out_specs=pl.BlockSpec((None, Fa, Fa), lambda p, k: (p, 0, 0)),
            scratch_shapes=[pltpu.VMEM((Fa, Fa), jnp.float32),
                            pltpu.VMEM((Fa, tile_n), jnp.float32)]),
        compiler_params=pltpu.CompilerParams(
            dimension_semantics=("parallel", "arbitrary")),
    )(obs2d, y_row)

    gram = jnp.sum(gram_parts, axis=0)
    xtx = gram[:F, :F]
    xty = gram[:F, F]
    w = _ridge_solve(xtx, xty)

    tile_r = 8
    for cand in (4096, 2048, 1024, 512, 256, 128, 64, 32, 16, 8):
        if N % cand == 0:
            tile_r = cand
            break
    ntr = N // tile_r
    wmat = jnp.zeros((8, 128), jnp.float32)
    wmat = wmat.at[0, :D].set(w[:D]).at[1, :D].set(w[D:2 * D])
    wmat = wmat.at[2, :4].set(
        jnp.stack([w[2 * D + 3], w[2 * D], w[2 * D + 1], w[2 * D + 2]]))

    val = pl.pallas_call(
        functools.partial(_value_kernel, bsz=B, tile_r=tile_r),
        out_shape=jax.ShapeDtypeStruct((N, 1), jnp.float32),
        grid_spec=pltpu.PrefetchScalarGridSpec(
            num_scalar_prefetch=0,
            grid=(ntr,),
            in_specs=[pl.BlockSpec((tile_r, D), lambda i: (i, 0)),
                      pl.BlockSpec((8, 128), lambda i: (0, 0))],
            out_specs=pl.BlockSpec((tile_r, 1), lambda i: (i, 0))),
        compiler_params=pltpu.CompilerParams(
            dimension_semantics=("parallel",)),
    )(obs2d, wmat)

    return val.reshape(S, B).T
